# Initial kernel scaffold; baseline (speedup 1.0000x reference)
import jax
import jax.numpy as jnp
from jax.experimental import pallas as pl


def kernel(x, W_router, W1, W2):
    raise NotImplementedError("write your pallas kernel here")


# speedup vs baseline: 1.2080x; 1.2080x over previous
import functools

import jax
import jax.numpy as jnp
from jax.experimental import pallas as pl
from jax.experimental.pallas import tpu as pltpu

T, D, E, H, TOP_K = 2048, 1024, 8, 4096, 2
LOGIT_CAP = 30.0
LB_COEFF = 0.01


def _router_kernel(x_ref, wr_ref, combine_ref, aux_ref):
    x = x_ref[:]
    wr = wr_ref[:]
    logits = jnp.dot(x, wr, preferred_element_type=jnp.float32)
    logits = LOGIT_CAP * jnp.tanh(logits / LOGIT_CAP)
    m = jnp.max(logits, axis=1, keepdims=True)
    p = jnp.exp(logits - m)
    probs = p / jnp.sum(p, axis=1, keepdims=True)

    iota = jax.lax.broadcasted_iota(jnp.int32, (T, E), 1)
    big = jnp.int32(E + 1)
    v1 = jnp.max(probs, axis=1, keepdims=True)
    i1 = jnp.min(jnp.where(probs == v1, iota, big), axis=1, keepdims=True)
    oh1 = (iota == i1).astype(jnp.float32)
    probs2 = jnp.where(iota == i1, -1.0, probs)
    v2 = jnp.max(probs2, axis=1, keepdims=True)
    i2 = jnp.min(jnp.where(probs2 == v2, iota, big), axis=1, keepdims=True)
    oh2 = (iota == i2).astype(jnp.float32)

    denom = v1 + v2 + 1e-9
    g1 = v1 / denom
    g2 = v2 / denom
    combine = oh1 * g1 + oh2 * g2
    combine_ref[:] = combine

    me = jnp.mean(probs, axis=0, keepdims=True)
    ce = jnp.mean(combine, axis=0, keepdims=True)
    aux_ref[0, 0] = LB_COEFF * E * TOP_K * jnp.sum(me * ce)


def _ffn_kernel(x_ref, w1_ref, w2_ref, combine_ref, y_ref, acc_ref):
    e = pl.program_id(0)
    hc = pl.program_id(1)
    xb = x_ref[:].astype(jnp.bfloat16)
    w1 = w1_ref[0].astype(jnp.bfloat16)
    h = jnp.dot(xb, w1, preferred_element_type=jnp.float32)
    h = jnp.square(jnp.maximum(h, 0.0)).astype(jnp.bfloat16)
    w2 = w2_ref[0].astype(jnp.bfloat16)
    part = jnp.dot(h, w2, preferred_element_type=jnp.float32)

    iota_e = jax.lax.broadcasted_iota(jnp.int32, (T, E), 1)
    c = jnp.sum(jnp.where(iota_e == e, combine_ref[:], 0.0), axis=1,
                keepdims=True)
    contrib = c * part

    @pl.when(jnp.logical_and(e == 0, hc == 0))
    def _init():
        acc_ref[:] = contrib

    @pl.when(jnp.logical_not(jnp.logical_and(e == 0, hc == 0)))
    def _acc():
        acc_ref[:] = acc_ref[:] + contrib

    @pl.when(jnp.logical_and(e == E - 1, hc == pl.num_programs(1) - 1))
    def _done():
        y_ref[:] = acc_ref[:]


def kernel(x, W_router, W1, W2):
    combine, aux = pl.pallas_call(
        _router_kernel,
        out_shape=(
            jax.ShapeDtypeStruct((T, E), jnp.float32),
            jax.ShapeDtypeStruct((1, 1), jnp.float32),
        ),
        in_specs=[
            pl.BlockSpec((T, D), lambda: (0, 0)),
            pl.BlockSpec((D, E), lambda: (0, 0)),
        ],
        out_specs=(
            pl.BlockSpec((T, E), lambda: (0, 0)),
            pl.BlockSpec((1, 1), lambda: (0, 0), memory_space=pltpu.SMEM),
        ),
    )(x, W_router)

    HC = 1024
    n_hc = H // HC
    y = pl.pallas_call(
        _ffn_kernel,
        grid=(E, n_hc),
        out_shape=jax.ShapeDtypeStruct((T, D), jnp.float32),
        in_specs=[
            pl.BlockSpec((T, D), lambda e, hc: (0, 0)),
            pl.BlockSpec((1, D, HC), lambda e, hc: (e, 0, hc)),
            pl.BlockSpec((1, HC, D), lambda e, hc: (e, hc, 0)),
            pl.BlockSpec((T, E), lambda e, hc: (0, 0)),
        ],
        out_specs=pl.BlockSpec((T, D), lambda e, hc: (0, 0)),
        scratch_shapes=[pltpu.VMEM((T, D), jnp.float32)],
        compiler_params=pltpu.CompilerParams(
            dimension_semantics=("arbitrary", "arbitrary"),
        ),
    )(x, W1, W2, combine)

    return y, aux.reshape(())


# speedup vs baseline: 1.4250x; 1.1797x over previous
import jax
import jax.numpy as jnp
from jax.experimental import pallas as pl
from jax.experimental.pallas import tpu as pltpu

T, D, E, H, TOP_K = 2048, 1024, 8, 4096, 2
LOGIT_CAP = 30.0
LB_COEFF = 0.01

TILE = 128
P = 5120
NT = P // TILE
SCH = 512
NSC = 16
HC = H // 2


def _router_kernel(x_ref, wr_ref, st_ref, te_ref, used_ref,
                   pos1_ref, pos2_ref, g1_ref, g2_ref, aux_ref):
    x = x_ref[:]
    logits = jnp.dot(x, wr_ref[:], preferred_element_type=jnp.float32)
    logits = LOGIT_CAP * jnp.tanh(logits / LOGIT_CAP)
    m = jnp.max(logits, axis=1, keepdims=True)
    p = jnp.exp(logits - m)
    probs = p / jnp.sum(p, axis=1, keepdims=True)

    iota_e = jax.lax.broadcasted_iota(jnp.int32, (T, E), 1)
    big = jnp.int32(E + 1)
    v1 = jnp.max(probs, axis=1, keepdims=True)
    i1 = jnp.min(jnp.where(probs == v1, iota_e, big), axis=1, keepdims=True)
    oh1 = (iota_e == i1).astype(jnp.float32)
    probs2 = jnp.where(iota_e == i1, -1.0, probs)
    v2 = jnp.max(probs2, axis=1, keepdims=True)
    i2 = jnp.min(jnp.where(probs2 == v2, iota_e, big), axis=1, keepdims=True)
    oh2 = (iota_e == i2).astype(jnp.float32)

    denom = v1 + v2 + 1e-9
    g1 = v1 / denom
    g2 = v2 / denom
    g1_ref[:] = g1
    g2_ref[:] = g2

    combine = oh1 * g1 + oh2 * g2
    me = jnp.mean(probs, axis=0, keepdims=True)
    ce = jnp.mean(combine, axis=0, keepdims=True)
    aux_ref[0, 0] = LB_COEFF * E * TOP_K * jnp.sum(me * ce)

    n1 = jnp.sum(oh1, axis=0, keepdims=True)
    n2 = jnp.sum(oh2, axis=0, keepdims=True)
    n = n1 + n2
    n_i = n.astype(jnp.int32)
    pc = (((n_i + TILE - 1) // TILE) * TILE).astype(jnp.float32)

    u8a = jax.lax.broadcasted_iota(jnp.int32, (E, E), 0)
    u8b = jax.lax.broadcasted_iota(jnp.int32, (E, E), 1)
    triu = (u8a < u8b).astype(jnp.float32)
    off = jnp.dot(pc, triu, preferred_element_type=jnp.float32)

    total = jnp.sum(pc)
    used_ref[0, 0] = (total.astype(jnp.int32)) // TILE

    lu = jnp.max(jnp.where(n_i > 0, jax.lax.broadcasted_iota(jnp.int32, (1, E), 1),
                           -1))
    starts = (jax.lax.broadcasted_iota(jnp.int32, (NT, 1), 0) * TILE
              ).astype(jnp.float32)
    cnt = jnp.sum((jnp.broadcast_to(off, (NT, E)) <= starts).astype(jnp.int32),
                  axis=1, keepdims=True)
    te_ref[:] = jnp.minimum(cnt - 1, lu)

    c_iota_a = jax.lax.broadcasted_iota(jnp.int32, (256, 256), 0)
    c_iota_b = jax.lax.broadcasted_iota(jnp.int32, (256, 256), 1)
    tril = (c_iota_b < c_iota_a).astype(jnp.float32)
    rank1_chunks = []
    rank2_chunks = []
    carry1 = jnp.zeros((1, E), jnp.float32)
    carry2 = jnp.zeros((1, E), jnp.float32)
    for c in range(T // 256):
        o1c = oh1[c * 256:(c + 1) * 256]
        o2c = oh2[c * 256:(c + 1) * 256]
        cum1 = jnp.dot(tril, o1c, preferred_element_type=jnp.float32) + carry1
        cum2 = jnp.dot(tril, o2c, preferred_element_type=jnp.float32) + carry2
        carry1 = carry1 + jnp.sum(o1c, axis=0, keepdims=True)
        carry2 = carry2 + jnp.sum(o2c, axis=0, keepdims=True)
        rank1_chunks.append(jnp.sum(o1c * cum1, axis=1, keepdims=True))
        rank2_chunks.append(jnp.sum(o2c * cum2, axis=1, keepdims=True))
    rank1 = jnp.concatenate(rank1_chunks, axis=0)
    rank2 = jnp.concatenate(rank2_chunks, axis=0)

    offb = jnp.broadcast_to(off, (T, E))
    base1 = jnp.sum(oh1 * offb, axis=1, keepdims=True)
    base2 = jnp.sum(oh2 * (offb + jnp.broadcast_to(n1, (T, E))), axis=1,
                    keepdims=True)
    pos1 = (base1 + rank1).astype(jnp.int32)
    pos2 = (base2 + rank2).astype(jnp.int32)
    pos1_ref[:] = pos1
    pos2_ref[:] = pos2

    tok = jax.lax.broadcasted_iota(jnp.int32, (1, T), 1)
    tok_hi = (tok // 256).astype(jnp.float32)
    tok_lo = (tok % 256).astype(jnp.float32)
    for c in range(NSC):
        j_iota = jax.lax.broadcasted_iota(jnp.int32, (T, SCH), 1) + c * SCH
        msk = ((pos1 == j_iota).astype(jnp.float32)
               + (pos2 == j_iota).astype(jnp.float32))
        hi = jnp.dot(tok_hi, msk, preferred_element_type=jnp.float32)
        lo = jnp.dot(tok_lo, msk, preferred_element_type=jnp.float32)
        st_ref[pl.ds(c, 1), :] = (hi.astype(jnp.int32) * 256
                                  + lo.astype(jnp.int32))


def _ffn_a_kernel(te_ref, used_ref, st_ref, x_ref, w1_ref, w2_ref,
                  xs_ref, a_ref, xbf_ref):
    i = pl.program_id(0)

    @pl.when(i == 0)
    def _cast():
        xbf_ref[:] = x_ref[:].astype(jnp.bfloat16)

    @pl.when(i < used_ref[0])
    def _compute():
        tok = st_ref[:]
        iota_t = jax.lax.broadcasted_iota(jnp.int32, (TILE, T), 1)
        oh = (tok == iota_t).astype(jnp.bfloat16)
        xs = jnp.dot(oh, xbf_ref[:], preferred_element_type=jnp.float32)
        xsb = xs.astype(jnp.bfloat16)
        xs_ref[:] = xsb
        h = jnp.dot(xsb, w1_ref[0].astype(jnp.bfloat16),
                    preferred_element_type=jnp.float32)
        h = jnp.square(jnp.maximum(h, 0.0)).astype(jnp.bfloat16)
        a_ref[:] = jnp.dot(h, w2_ref[0].astype(jnp.bfloat16),
                           preferred_element_type=jnp.float32)

    @pl.when(i >= used_ref[0])
    def _pad():
        xs_ref[:] = jnp.zeros_like(xs_ref)
        a_ref[:] = jnp.zeros_like(a_ref)


def _ffn_b_kernel(te_ref, used_ref, xs_ref, a_ref, w1_ref, w2_ref, out_ref):
    i = pl.program_id(0)

    @pl.when(i < used_ref[0])
    def _compute():
        h = jnp.dot(xs_ref[:], w1_ref[0].astype(jnp.bfloat16),
                    preferred_element_type=jnp.float32)
        h = jnp.square(jnp.maximum(h, 0.0)).astype(jnp.bfloat16)
        out_ref[:] = a_ref[:] + jnp.dot(h, w2_ref[0].astype(jnp.bfloat16),
                                        preferred_element_type=jnp.float32)

    @pl.when(i >= used_ref[0])
    def _pad():
        out_ref[:] = jnp.zeros_like(out_ref)


def _combine_kernel(pos1_ref, pos2_ref, g1_ref, g2_ref, os_ref, y_ref,
                    osb_ref):
    i = pl.program_id(0)

    @pl.when(i == 0)
    def _cast():
        osb_ref[:] = os_ref[:].astype(jnp.bfloat16)

    BT = T // 8
    j_iota = jax.lax.broadcasted_iota(jnp.int32, (BT, P), 1)
    msk = (jnp.where(pos1_ref[:] == j_iota, g1_ref[:], 0.0)
           + jnp.where(pos2_ref[:] == j_iota, g2_ref[:], 0.0))
    y_ref[:] = jnp.dot(msk.astype(jnp.bfloat16), osb_ref[:],
                       preferred_element_type=jnp.float32)


def kernel(x, W_router, W1, W2):
    st, te, used, pos1, pos2, g1, g2, aux = pl.pallas_call(
        _router_kernel,
        out_shape=(
            jax.ShapeDtypeStruct((NSC, SCH), jnp.int32),
            jax.ShapeDtypeStruct((NT, 1), jnp.int32),
            jax.ShapeDtypeStruct((1, 1), jnp.int32),
            jax.ShapeDtypeStruct((T, 1), jnp.int32),
            jax.ShapeDtypeStruct((T, 1), jnp.int32),
            jax.ShapeDtypeStruct((T, 1), jnp.float32),
            jax.ShapeDtypeStruct((T, 1), jnp.float32),
            jax.ShapeDtypeStruct((1, 1), jnp.float32),
        ),
        in_specs=[
            pl.BlockSpec((T, D), lambda: (0, 0)),
            pl.BlockSpec((D, E), lambda: (0, 0)),
        ],
        out_specs=(
            pl.BlockSpec((NSC, SCH), lambda: (0, 0)),
            pl.BlockSpec((NT, 1), lambda: (0, 0)),
            pl.BlockSpec((1, 1), lambda: (0, 0), memory_space=pltpu.SMEM),
            pl.BlockSpec((T, 1), lambda: (0, 0)),
            pl.BlockSpec((T, 1), lambda: (0, 0)),
            pl.BlockSpec((T, 1), lambda: (0, 0)),
            pl.BlockSpec((T, 1), lambda: (0, 0)),
            pl.BlockSpec((1, 1), lambda: (0, 0), memory_space=pltpu.SMEM),
        ),
    )(x, W_router)

    te_r = te.reshape(NT)
    used_r = used.reshape(1)
    st_r = st.reshape(NSC * SCH, 1)[:P]

    xs, a_half = pl.pallas_call(
        _ffn_a_kernel,
        grid_spec=pltpu.PrefetchScalarGridSpec(
            num_scalar_prefetch=2,
            grid=(NT,),
            in_specs=[
                pl.BlockSpec((TILE, 1), lambda i, te, u: (i, 0)),
                pl.BlockSpec((T, D), lambda i, te, u: (0, 0)),
                pl.BlockSpec((1, D, HC), lambda i, te, u: (te[i], 0, 0)),
                pl.BlockSpec((1, HC, D), lambda i, te, u: (te[i], 0, 0)),
            ],
            out_specs=(
                pl.BlockSpec((TILE, D), lambda i, te, u: (i, 0)),
                pl.BlockSpec((TILE, D), lambda i, te, u: (i, 0)),
            ),
            scratch_shapes=[pltpu.VMEM((T, D), jnp.bfloat16)],
        ),
        out_shape=(
            jax.ShapeDtypeStruct((P, D), jnp.bfloat16),
            jax.ShapeDtypeStruct((P, D), jnp.float32),
        ),
        compiler_params=pltpu.CompilerParams(
            dimension_semantics=("arbitrary",),
        ),
    )(te_r, used_r, st_r, x, W1, W2)

    out_sorted = pl.pallas_call(
        _ffn_b_kernel,
        grid_spec=pltpu.PrefetchScalarGridSpec(
            num_scalar_prefetch=2,
            grid=(NT,),
            in_specs=[
                pl.BlockSpec((TILE, D), lambda i, te, u: (i, 0)),
                pl.BlockSpec((TILE, D), lambda i, te, u: (i, 0)),
                pl.BlockSpec((1, D, HC), lambda i, te, u: (te[i], 0, 1)),
                pl.BlockSpec((1, HC, D), lambda i, te, u: (te[i], 1, 0)),
            ],
            out_specs=pl.BlockSpec((TILE, D), lambda i, te, u: (i, 0)),
        ),
        out_shape=jax.ShapeDtypeStruct((P, D), jnp.float32),
        compiler_params=pltpu.CompilerParams(
            dimension_semantics=("arbitrary",),
        ),
    )(te_r, used_r, xs, a_half, W1, W2)

    BT = T // 8
    y = pl.pallas_call(
        _combine_kernel,
        grid=(8,),
        out_shape=jax.ShapeDtypeStruct((T, D), jnp.float32),
        in_specs=[
            pl.BlockSpec((BT, 1), lambda i: (i, 0)),
            pl.BlockSpec((BT, 1), lambda i: (i, 0)),
            pl.BlockSpec((BT, 1), lambda i: (i, 0)),
            pl.BlockSpec((BT, 1), lambda i: (i, 0)),
            pl.BlockSpec((P, D), lambda i: (0, 0)),
        ],
        out_specs=pl.BlockSpec((BT, D), lambda i: (i, 0)),
        scratch_shapes=[pltpu.VMEM((P, D), jnp.bfloat16)],
        compiler_params=pltpu.CompilerParams(
            dimension_semantics=("arbitrary",),
        ),
    )(pos1, pos2, g1, g2, out_sorted)

    return y, aux.reshape(())
